# baseline (speedup 1.0000x reference)
import functools
import math

import jax
import jax.numpy as jnp
from jax import lax
from jax.experimental import pallas as pl
from jax.experimental.pallas import tpu as pltpu
from jax.experimental.pallas import tpu_sc as plsc

_NE = 4096
_ZD = 128
_B = 8192
_XD = 5
_PREC = 10.0
_BETA = 1.0
_C0 = -0.5 * _XD * math.log(2.0 * math.pi / _PREC)

_BT = 2048
_NC, _NS = 2, 16
_NW = _NC * _NS
_BW = _B // _NW


def _relu(v):
    return jnp.maximum(v, 0.0)


def _dot(a, b):
    return jax.lax.dot_general(
        a, b, (((1,), (0,)), ((), ())), preferred_element_type=jnp.float32)


def _dott(a, b):
    return jax.lax.dot_general(
        a, b, (((1,), (1,)), ((), ())), preferred_element_type=jnp.float32)


def _main_body(x_ref, emb_ref,
               w1_ref, b1_ref, w2_ref, b2_ref, w3_ref, b3_ref, w4_ref, b4_ref,
               w5_ref, b5_ref, w6_ref, b6_ref, w7_ref, b7_ref, w8_ref, b8_ref,
               mi_ref, vq_ref, rt_ref, ea_ref, e2_ref):
    i = pl.program_id(0)

    @pl.when(i == 0)
    def _init():
        e = emb_ref[...]
        ea_ref[...] = 2.0 * jnp.transpose(e, (1, 0))
        ea = ea_ref[...]
        e2_ref[...] = 1.0 - 0.25 * jnp.sum(ea * ea, axis=0, keepdims=True)
        h = _relu(_dott(e, w5_ref[...]) + b5_ref[...])
        h = _relu(_dott(h, w6_ref[...]) + b6_ref[...])
        h = _relu(_dott(h, w7_ref[...]) + b7_ref[...])
        rt_ref[...] = _dott(h, w8_ref[...]) + b8_ref[...]
        vq_ref[...] = jnp.zeros_like(vq_ref)

    xb = x_ref[...]
    h = _relu(_dott(xb, w1_ref[...]) + b1_ref[...])
    h = _relu(_dott(h, w2_ref[...]) + b2_ref[...])
    h = _relu(_dott(h, w3_ref[...]) + b3_ref[...])
    z = _dott(h, w4_ref[...]) + b4_ref[...]

    tc = _dot(z, ea_ref[...]) + e2_ref[...]
    keys = lax.bitcast_convert_type(
        (lax.bitcast_convert_type(tc, jnp.int32) & jnp.int32(~0xFFF))
        | lax.broadcasted_iota(jnp.int32, tc.shape, 1), jnp.float32)
    kmax = jnp.max(keys, axis=1)
    ki = lax.bitcast_convert_type(kmax, jnp.int32)
    mi_ref[...] = ki & jnp.int32(0xFFF)
    mn = lax.bitcast_convert_type(ki & jnp.int32(~0xFFF), jnp.float32) - 1.0

    part = jnp.sum(z * z) - jnp.sum(mn)
    vq_ref[...] = vq_ref[...] + part


def _sc_gather_body(rt_hbm, idx_hbm, x_hbm, out_hbm,
                    tab_sh, idx_v, rows_v, x_v, acc_v, sem):
    sid = lax.axis_index("s")
    wid = sid * _NC + lax.axis_index("c")
    base = wid * _BW
    @pl.when(sid == 0)
    def _stage():
        pltpu.sync_copy(rt_hbm, tab_sh)
    pltpu.sync_copy(idx_hbm.at[pl.ds(base, _BW)], idx_v)
    pltpu.sync_copy(x_hbm.at[pl.ds(base, _BW)], x_v)
    plsc.subcore_barrier()
    cp = pltpu.async_copy(tab_sh.at[idx_v], rows_v, sem)
    cp.wait()

    def body(i, acc):
        d = x_v[i, :] - rows_v[i, :]
        return acc + d * d

    acc_v[...] = lax.fori_loop(0, _BW, body, jnp.zeros((16,), jnp.float32))
    pltpu.sync_copy(acc_v, out_hbm.at[wid, pl.ds(0, 16)])


def _final_body(vq_ref, sq_ref, o_ref):
    s = jnp.sum(sq_ref[:, 0:16])
    o_ref[...] = (1.0 + _BETA) * vq_ref[...] + (0.5 * _PREC * s - _B * _C0)


def kernel(x, emb, W1, b1, W2, b2, W3, b3, W4, b4,
           W5, b5, W6, b6, W7, b7, W8, b8):
    f32 = jnp.float32
    row = lambda v: v.reshape(1, -1)
    w1p = jnp.zeros((32, 16), f32).at[:16, :_XD].set(W1)
    b1p = jnp.zeros((1, 32), f32).at[:, :16].set(b1[None, :])
    w2p = jnp.zeros((32, 32), f32).at[:, :16].set(W2)
    w3p = jnp.zeros((32, 32), f32).at[:16, :].set(W3)
    b3p = jnp.zeros((1, 32), f32).at[:, :16].set(b3[None, :])
    w4p = jnp.zeros((_ZD, 32), f32).at[:, :16].set(W4)
    w5p = jnp.zeros((32, _ZD), f32).at[:16, :].set(W5)
    b5p = jnp.zeros((1, 32), f32).at[:, :16].set(b5[None, :])
    w6p = jnp.zeros((32, 32), f32).at[:, :16].set(W6)
    w7p = jnp.zeros((32, 32), f32).at[:16, :].set(W7)
    b7p = jnp.zeros((1, 32), f32).at[:, :16].set(b7[None, :])
    w8p = jnp.zeros((16, 32), f32).at[:_XD, :16].set(W8)
    b8p = jnp.zeros((1, 16), f32).at[:, :_XD].set(b8[None, :])

    n_t = _B // _BT
    full = lambda a: pl.BlockSpec(a.shape, lambda i: (0,) * a.ndim)

    weights = [w1p, b1p, w2p, row(b2), w3p, b3p, w4p, row(b4),
               w5p, b5p, w6p, row(b6), w7p, b7p, w8p, b8p]

    xp = jnp.zeros((_B, 16), f32).at[:, :_XD].set(x)

    mi, vq, rtab = pl.pallas_call(
        _main_body,
        grid=(n_t,),
        in_specs=[pl.BlockSpec((_BT, 16), lambda i: (i, 0)),
                  full(emb)] + [full(w) for w in weights],
        out_specs=[pl.BlockSpec((_BT,), lambda i: (i,)),
                   pl.BlockSpec((1, 1), lambda i: (0, 0)),
                   pl.BlockSpec((_NE, 16), lambda i: (0, 0))],
        out_shape=[jax.ShapeDtypeStruct((_B,), jnp.int32),
                   jax.ShapeDtypeStruct((1, 1), f32),
                   jax.ShapeDtypeStruct((_NE, 16), f32)],
        scratch_shapes=[pltpu.VMEM((_ZD, _NE), f32),
                        pltpu.VMEM((1, _NE), f32)],
    )(xp, emb, *weights)

    mesh = plsc.VectorSubcoreMesh(core_axis_name="c", subcore_axis_name="s",
                                  num_cores=_NC, num_subcores=_NS)
    sq = pl.kernel(
        _sc_gather_body,
        mesh=mesh,
        out_type=jax.ShapeDtypeStruct((_NW, 128), f32),
        scratch_types=[pltpu.VMEM_SHARED((_NE, 16), f32),
                       pltpu.VMEM((_BW,), jnp.int32),
                       pltpu.VMEM((_BW, 16), f32),
                       pltpu.VMEM((_BW, 16), f32),
                       pltpu.VMEM((16,), f32),
                       pltpu.SemaphoreType.DMA],
        compiler_params=pltpu.CompilerParams(use_tc_tiling_on_sc=False),
    )(rtab, mi, xp)

    loss = pl.pallas_call(
        _final_body,
        in_specs=[pl.BlockSpec(vq.shape, lambda: (0, 0)),
                  pl.BlockSpec(sq.shape, lambda: (0, 0))],
        out_specs=pl.BlockSpec((1, 1), lambda: (0, 0)),
        out_shape=jax.ShapeDtypeStruct((1, 1), f32),
    )(vq, sq)
    return loss[0, 0]


# speedup vs baseline: 1.0937x; 1.0937x over previous
import functools
import math

import jax
import jax.numpy as jnp
from jax import lax
from jax.experimental import pallas as pl
from jax.experimental.pallas import tpu as pltpu
from jax.experimental.pallas import tpu_sc as plsc

_NE = 4096
_ZD = 128
_B = 8192
_XD = 5
_PREC = 10.0
_BETA = 1.0
_C0 = -0.5 * _XD * math.log(2.0 * math.pi / _PREC)

_BT = 2048
_NC, _NS = 2, 16
_NW = _NC * _NS
_BW = _B // _NW


def _relu(v):
    return jnp.maximum(v, 0.0)


def _dot(a, b):
    return jax.lax.dot_general(
        a, b, (((1,), (0,)), ((), ())), preferred_element_type=jnp.float32)


def _dott(a, b):
    return jax.lax.dot_general(
        a, b, (((1,), (1,)), ((), ())), preferred_element_type=jnp.float32)


def _main_body(x_ref, emb_ref,
               w1_ref, b1_ref, w2_ref, b2_ref, w3_ref, b3_ref, w4_ref, b4_ref,
               w5_ref, b5_ref, w6_ref, b6_ref, w7_ref, b7_ref, w8_ref, b8_ref,
               mi_ref, vq_ref, rt_ref, ea_ref, e2_ref):
    i = pl.program_id(0)

    @pl.when(i == 0)
    def _init():
        e = emb_ref[...]
        ea_ref[...] = 2.0 * jnp.transpose(e, (1, 0))
        ea = ea_ref[...]
        e2_ref[...] = 1.0 - 0.25 * jnp.sum(ea * ea, axis=0, keepdims=True)
        h = _relu(_dott(e, w5_ref[...]) + b5_ref[...])
        h = _relu(_dott(h, w6_ref[...]) + b6_ref[...])
        h = _relu(_dott(h, w7_ref[...]) + b7_ref[...])
        rt_ref[...] = _dott(h, w8_ref[...]) + b8_ref[...]
        vq_ref[...] = jnp.zeros_like(vq_ref)

    xb = x_ref[...]
    h = _relu(_dott(xb, w1_ref[...]) + b1_ref[...])
    h = _relu(_dott(h, w2_ref[...]) + b2_ref[...])
    h = _relu(_dott(h, w3_ref[...]) + b3_ref[...])
    z = _dott(h, w4_ref[...]) + b4_ref[...]

    tc = _dot(z, ea_ref[...]) + e2_ref[...]
    keys = lax.bitcast_convert_type(
        (lax.bitcast_convert_type(tc, jnp.int32) & jnp.int32(~0xFFF))
        | lax.broadcasted_iota(jnp.int32, tc.shape, 1), jnp.float32)
    kmax = jnp.max(keys, axis=1)
    ki = lax.bitcast_convert_type(kmax, jnp.int32)
    mi_ref[...] = ki & jnp.int32(0xFFF)
    mn = lax.bitcast_convert_type(ki & jnp.int32(~0xFFF), jnp.float32) - 1.0

    part = jnp.sum(z * z) - jnp.sum(mn)
    vq_ref[...] = vq_ref[...] + part


def _sc_gather_body(rt_hbm, idx_hbm, x_hbm, out_hbm,
                    tab_sh, idx_v, rows_v, x_v, acc_v, sem):
    sid = lax.axis_index("s")
    wid = sid * _NC + lax.axis_index("c")
    base = wid * _BW
    @pl.when(sid == 0)
    def _stage():
        pltpu.sync_copy(rt_hbm, tab_sh)
    pltpu.sync_copy(idx_hbm.at[pl.ds(base, _BW)], idx_v)
    pltpu.sync_copy(x_hbm.at[pl.ds(base, _BW)], x_v)
    plsc.subcore_barrier()
    cp = pltpu.async_copy(tab_sh.at[idx_v], rows_v, sem)
    cp.wait()

    def body(i, acc):
        d = x_v[i, :] - rows_v[i, :]
        return acc + d * d

    acc_v[...] = lax.fori_loop(0, _BW, body, jnp.zeros((16,), jnp.float32))
    pltpu.sync_copy(acc_v, out_hbm.at[wid, pl.ds(0, 16)])


def _final_body(vq_ref, sq_ref, o_ref):
    s = jnp.sum(sq_ref[:, 0:16])
    o_ref[...] = (1.0 + _BETA) * vq_ref[...] + (0.5 * _PREC * s - _B * _C0)


def kernel(x, emb, W1, b1, W2, b2, W3, b3, W4, b4,
           W5, b5, W6, b6, W7, b7, W8, b8):
    f32 = jnp.float32
    row = lambda v: v.reshape(1, -1)
    w1p = jnp.zeros((16, 16), f32).at[:, :_XD].set(W1)
    w8p = jnp.zeros((16, 16), f32).at[:_XD, :].set(W8)
    b8p = jnp.zeros((1, 16), f32).at[:, :_XD].set(b8[None, :])

    n_t = _B // _BT
    full = lambda a: pl.BlockSpec(a.shape, lambda i: (0,) * a.ndim)

    weights = [w1p, row(b1), W2, row(b2), W3, row(b3), W4, row(b4),
               W5, row(b5), W6, row(b6), W7, row(b7), w8p, b8p]

    xp = jnp.zeros((_B, 16), f32).at[:, :_XD].set(x)

    mi, vq, rtab = pl.pallas_call(
        _main_body,
        grid=(n_t,),
        in_specs=[pl.BlockSpec((_BT, 16), lambda i: (i, 0)),
                  full(emb)] + [full(w) for w in weights],
        out_specs=[pl.BlockSpec((_BT,), lambda i: (i,)),
                   pl.BlockSpec((1, 1), lambda i: (0, 0)),
                   pl.BlockSpec((_NE, 16), lambda i: (0, 0))],
        out_shape=[jax.ShapeDtypeStruct((_B,), jnp.int32),
                   jax.ShapeDtypeStruct((1, 1), f32),
                   jax.ShapeDtypeStruct((_NE, 16), f32)],
        scratch_shapes=[pltpu.VMEM((_ZD, _NE), f32),
                        pltpu.VMEM((1, _NE), f32)],
    )(xp, emb, *weights)

    mesh = plsc.VectorSubcoreMesh(core_axis_name="c", subcore_axis_name="s",
                                  num_cores=_NC, num_subcores=_NS)
    sq = pl.kernel(
        _sc_gather_body,
        mesh=mesh,
        out_type=jax.ShapeDtypeStruct((_NW, 128), f32),
        scratch_types=[pltpu.VMEM_SHARED((_NE, 16), f32),
                       pltpu.VMEM((_BW,), jnp.int32),
                       pltpu.VMEM((_BW, 16), f32),
                       pltpu.VMEM((_BW, 16), f32),
                       pltpu.VMEM((16,), f32),
                       pltpu.SemaphoreType.DMA],
        compiler_params=pltpu.CompilerParams(use_tc_tiling_on_sc=False),
    )(rtab, mi, xp)

    loss = pl.pallas_call(
        _final_body,
        in_specs=[pl.BlockSpec(vq.shape, lambda: (0, 0)),
                  pl.BlockSpec(sq.shape, lambda: (0, 0))],
        out_specs=pl.BlockSpec((1, 1), lambda: (0, 0)),
        out_shape=jax.ShapeDtypeStruct((1, 1), f32),
    )(vq, sq)
    return loss[0, 0]


# speedup vs baseline: 1.1001x; 1.0058x over previous
import functools
import math

import jax
import jax.numpy as jnp
from jax import lax
from jax.experimental import pallas as pl
from jax.experimental.pallas import tpu as pltpu
from jax.experimental.pallas import tpu_sc as plsc

_NE = 4096
_ZD = 128
_B = 8192
_XD = 5
_PREC = 10.0
_BETA = 1.0
_C0 = -0.5 * _XD * math.log(2.0 * math.pi / _PREC)

_BT = 2048
_NC, _NS = 2, 16
_NW = _NC * _NS
_BW = _B // _NW


def _relu(v):
    return jnp.maximum(v, 0.0)


def _dot(a, b):
    return jax.lax.dot_general(
        a, b, (((1,), (0,)), ((), ())), preferred_element_type=jnp.float32)


def _dott(a, b):
    return jax.lax.dot_general(
        a, b, (((1,), (1,)), ((), ())), preferred_element_type=jnp.float32)


def _main_body(x_ref, emb_ref,
               w1_ref, b1_ref, w2_ref, b2_ref, w3_ref, b3_ref, w4_ref, b4_ref,
               w5_ref, b5_ref, w6_ref, b6_ref, w7_ref, b7_ref, w8_ref, b8_ref,
               mi_ref, vq_ref, rt_ref, ea_ref, e2_ref, col_ref):
    i = pl.program_id(0)

    @pl.when(i == 0)
    def _init():
        e = emb_ref[...]
        ea = 2.0 * jnp.transpose(e, (1, 0))
        ea_ref[...] = ea.astype(jnp.bfloat16)
        e2_ref[...] = 1.0 - 0.25 * jnp.sum(ea * ea, axis=0, keepdims=True)
        col_ref[...] = lax.broadcasted_iota(jnp.int32, (1, _NE), 1)
        h = _relu(_dott(e, w5_ref[...]) + b5_ref[...])
        h = _relu(_dott(h, w6_ref[...]) + b6_ref[...])
        h = _relu(_dott(h, w7_ref[...]) + b7_ref[...])
        rt_ref[...] = _dott(h, w8_ref[...]) + b8_ref[...]
        vq_ref[...] = jnp.zeros_like(vq_ref)

    xb = x_ref[...]
    h = _relu(_dott(xb, w1_ref[...]) + b1_ref[...])
    h = _relu(_dott(h, w2_ref[...]) + b2_ref[...])
    h = _relu(_dott(h, w3_ref[...]) + b3_ref[...])
    z = _dott(h, w4_ref[...]) + b4_ref[...]

    tc = _dot(z.astype(jnp.bfloat16), ea_ref[...]) + e2_ref[...]
    keys = lax.bitcast_convert_type(
        (lax.bitcast_convert_type(tc, jnp.int32) & jnp.int32(~0xFFF))
        | col_ref[...], jnp.float32)
    kmax = jnp.max(keys, axis=1)
    ki = lax.bitcast_convert_type(kmax, jnp.int32)
    mi_ref[...] = ki & jnp.int32(0xFFF)
    mn = lax.bitcast_convert_type(ki & jnp.int32(~0xFFF), jnp.float32) - 1.0

    part = jnp.sum(z * z) - jnp.sum(mn)
    vq_ref[...] = vq_ref[...] + part


def _sc_gather_body(rt_hbm, idx_hbm, x_hbm, out_hbm,
                    tab_sh, idx_v, rows_v, x_v, acc_v, sem):
    sid = lax.axis_index("s")
    wid = sid * _NC + lax.axis_index("c")
    base = wid * _BW
    @pl.when(sid == 0)
    def _stage():
        pltpu.sync_copy(rt_hbm, tab_sh)
    pltpu.sync_copy(idx_hbm.at[pl.ds(base, _BW)], idx_v)
    pltpu.sync_copy(x_hbm.at[pl.ds(base, _BW)], x_v)
    plsc.subcore_barrier()
    cp = pltpu.async_copy(tab_sh.at[idx_v], rows_v, sem)
    cp.wait()

    def body(i, acc):
        d = x_v[i, :] - rows_v[i, :]
        return acc + d * d

    acc_v[...] = lax.fori_loop(0, _BW, body, jnp.zeros((16,), jnp.float32))
    pltpu.sync_copy(acc_v, out_hbm.at[wid, pl.ds(0, 16)])


def _final_body(vq_ref, sq_ref, o_ref):
    s = jnp.sum(sq_ref[:, 0:16])
    o_ref[...] = (1.0 + _BETA) * vq_ref[...] + (0.5 * _PREC * s - _B * _C0)


def kernel(x, emb, W1, b1, W2, b2, W3, b3, W4, b4,
           W5, b5, W6, b6, W7, b7, W8, b8):
    f32 = jnp.float32
    row = lambda v: v.reshape(1, -1)
    w1p = jnp.zeros((16, 16), f32).at[:, :_XD].set(W1)
    w8p = jnp.zeros((16, 16), f32).at[:_XD, :].set(W8)
    b8p = jnp.zeros((1, 16), f32).at[:, :_XD].set(b8[None, :])

    n_t = _B // _BT
    full = lambda a: pl.BlockSpec(a.shape, lambda i: (0,) * a.ndim)

    weights = [w1p, row(b1), W2, row(b2), W3, row(b3), W4, row(b4),
               W5, row(b5), W6, row(b6), W7, row(b7), w8p, b8p]

    xp = jnp.zeros((_B, 16), f32).at[:, :_XD].set(x)

    mi, vq, rtab = pl.pallas_call(
        _main_body,
        grid=(n_t,),
        in_specs=[pl.BlockSpec((_BT, 16), lambda i: (i, 0)),
                  full(emb)] + [full(w) for w in weights],
        out_specs=[pl.BlockSpec((_BT,), lambda i: (i,)),
                   pl.BlockSpec((1, 1), lambda i: (0, 0)),
                   pl.BlockSpec((_NE, 16), lambda i: (0, 0))],
        out_shape=[jax.ShapeDtypeStruct((_B,), jnp.int32),
                   jax.ShapeDtypeStruct((1, 1), f32),
                   jax.ShapeDtypeStruct((_NE, 16), f32)],
        scratch_shapes=[pltpu.VMEM((_ZD, _NE), jnp.bfloat16),
                        pltpu.VMEM((1, _NE), f32),
                        pltpu.VMEM((1, _NE), jnp.int32)],
    )(xp, emb, *weights)

    mesh = plsc.VectorSubcoreMesh(core_axis_name="c", subcore_axis_name="s",
                                  num_cores=_NC, num_subcores=_NS)
    sq = pl.kernel(
        _sc_gather_body,
        mesh=mesh,
        out_type=jax.ShapeDtypeStruct((_NW, 128), f32),
        scratch_types=[pltpu.VMEM_SHARED((_NE, 16), f32),
                       pltpu.VMEM((_BW,), jnp.int32),
                       pltpu.VMEM((_BW, 16), f32),
                       pltpu.VMEM((_BW, 16), f32),
                       pltpu.VMEM((16,), f32),
                       pltpu.SemaphoreType.DMA],
        compiler_params=pltpu.CompilerParams(use_tc_tiling_on_sc=False),
    )(rtab, mi, xp)

    loss = pl.pallas_call(
        _final_body,
        in_specs=[pl.BlockSpec(vq.shape, lambda: (0, 0)),
                  pl.BlockSpec(sq.shape, lambda: (0, 0))],
        out_specs=pl.BlockSpec((1, 1), lambda: (0, 0)),
        out_shape=jax.ShapeDtypeStruct((1, 1), f32),
    )(vq, sq)
    return loss[0, 0]
